# baseline (speedup 1.0000x reference)
import functools

import jax
import jax.numpy as jnp
from jax import lax
from jax.experimental import pallas as pl
from jax.experimental.pallas import tpu as pltpu
from jax.experimental.pallas import tpu_sc as plsc

_CHUNK = 128
_NBUF = 4
_L = 16


def _lookup(idx2, table, v, d):
    n_rows, chunk = idx2.shape
    info = plsc.get_sparse_core_info()
    nw = info.num_cores * info.num_subcores
    n_chunks = n_rows // nw
    b_per_w = n_chunks * chunk
    nbuf = min(_NBUF, n_chunks)
    n_col = d // _L

    mesh = plsc.VectorSubcoreMesh(core_axis_name="c", subcore_axis_name="s")

    @functools.partial(
        pl.kernel,
        mesh=mesh,
        compiler_params=pltpu.CompilerParams(needs_layout_passes=False),
        out_type=jax.ShapeDtypeStruct((n_rows * chunk, d), jnp.float32),
        scratch_types=(
            [pltpu.VMEM((v * d,), jnp.float32)]
            + [pltpu.VMEM((n_chunks, chunk), jnp.int32)]
            + [pltpu.VMEM((chunk, d), jnp.float32) for _ in range(nbuf)]
            + [pltpu.SemaphoreType.DMA for _ in range(nbuf)]
        ),
    )
    def k(table_hbm, idx_hbm, out_hbm, table_v, idx_v, *rest):
        bufs = rest[:nbuf]
        sems_s = rest[nbuf : 2 * nbuf]
        wid = lax.axis_index("s") * info.num_cores + lax.axis_index("c")
        pltpu.sync_copy(table_hbm, table_v)
        pltpu.sync_copy(idx_hbm.at[pl.ds(wid * n_chunks, n_chunks)], idx_v)

        cols = [jnp.arange(_L, dtype=jnp.int32) + _L * j for j in range(n_col)]
        lane = [jnp.full((_L,), r, jnp.int32) for r in range(_L)]
        out_base = wid * b_per_w

        scat = {}
        for c in range(n_chunks):
            b = c % nbuf
            if c >= nbuf:
                scat[c - nbuf].wait()
            buf = bufs[b]

            def group_body(g, _, c=c, buf=buf):
                idxvec = idx_v[c, pl.ds(g * _L, _L)]
                for r in range(_L):
                    iv = idxvec.at[lane[r]].get(mode="promise_in_bounds")
                    ivd = iv * d
                    row = g * _L + r
                    for j in range(n_col):
                        vals = plsc.load_gather(table_v, [ivd + cols[j]])
                        buf[row, pl.ds(j * _L, _L)] = vals
                return 0

            lax.fori_loop(0, chunk // _L, group_body, 0)
            scat[c] = pltpu.async_copy(
                buf, out_hbm.at[pl.ds(out_base + c * chunk, chunk)], sems_s[b]
            )
        for c in range(n_chunks - nbuf, n_chunks):
            scat[c].wait()

    return k(table, idx2)


def kernel(input, table):
    v, d = table.shape
    idx = input.reshape(-1).astype(jnp.int32)
    idx2 = idx.reshape(-1, _CHUNK)
    out = _lookup(idx2, table.astype(jnp.float32).reshape(-1), v, d)
    return out.reshape(input.shape + (d,))


# speedup vs baseline: 1.1094x; 1.1094x over previous
import functools

import jax
import jax.numpy as jnp
from jax import lax
from jax.experimental import pallas as pl
from jax.experimental.pallas import tpu as pltpu
from jax.experimental.pallas import tpu_sc as plsc

_CHUNK = 128
_NBUF = 4
_L = 16


def _lookup(idx2, table, v, d):
    n_rows, chunk = idx2.shape
    info = plsc.get_sparse_core_info()
    nw = info.num_cores * info.num_subcores
    n_chunks = n_rows // nw
    b_per_w = n_chunks * chunk
    nbuf = min(_NBUF, n_chunks)
    n_col = d // _L
    n_grp = chunk // _L

    mesh = plsc.VectorSubcoreMesh(core_axis_name="c", subcore_axis_name="s")

    @functools.partial(
        pl.kernel,
        mesh=mesh,
        compiler_params=pltpu.CompilerParams(needs_layout_passes=False),
        out_type=jax.ShapeDtypeStruct((n_rows * chunk, d), jnp.float32),
        scratch_types=(
            [pltpu.VMEM((v * d,), jnp.float32)]
            + [pltpu.VMEM((n_chunks, chunk), jnp.int32)]
            + [pltpu.VMEM((nbuf * chunk, d), jnp.float32)]
            + [pltpu.SemaphoreType.DMA((nbuf,))]
        ),
    )
    def k(table_hbm, idx_hbm, out_hbm, table_v, idx_v, buf, sems):
        wid = lax.axis_index("s") * info.num_cores + lax.axis_index("c")
        pltpu.sync_copy(table_hbm, table_v)
        pltpu.sync_copy(idx_hbm.at[pl.ds(wid * n_chunks, n_chunks)], idx_v)

        cols = [jnp.arange(_L, dtype=jnp.int32) + _L * j for j in range(n_col)]
        lane = [jnp.full((_L,), r, jnp.int32) for r in range(_L)]
        out_base = wid * b_per_w

        def chunk_body(c, _):
            b = lax.rem(c, nbuf)
            slot = b * chunk

            @pl.when(c >= nbuf)
            def _wait_slot():
                pltpu.make_async_copy(
                    buf.at[pl.ds(slot, chunk)],
                    out_hbm.at[pl.ds(out_base, chunk)],
                    sems.at[b],
                ).wait()

            def group_body(g):
                idxvec = idx_v[c, pl.ds(g * _L, _L)]
                row0 = slot + g * _L
                for r in range(_L):
                    iv = idxvec.at[lane[r]].get(mode="promise_in_bounds")
                    ivd = iv * d
                    for j in range(n_col):
                        vals = plsc.load_gather(table_v, [ivd + cols[j]])
                        buf[row0 + r, pl.ds(j * _L, _L)] = vals

            plsc.parallel_loop(0, n_grp, unroll=2)(group_body)
            pltpu.async_copy(
                buf.at[pl.ds(slot, chunk)],
                out_hbm.at[pl.ds(out_base + c * chunk, chunk)],
                sems.at[b],
            )
            return 0

        lax.fori_loop(0, n_chunks, chunk_body, 0)
        for b in range(nbuf):
            pltpu.make_async_copy(
                buf.at[pl.ds(b * chunk, chunk)],
                out_hbm.at[pl.ds(out_base, chunk)],
                sems.at[b],
            ).wait()

    return k(table, idx2)


def kernel(input, table):
    v, d = table.shape
    idx = input.reshape(-1).astype(jnp.int32)
    idx2 = idx.reshape(-1, _CHUNK)
    out = _lookup(idx2, table.astype(jnp.float32).reshape(-1), v, d)
    return out.reshape(input.shape + (d,))


# speedup vs baseline: 1.4714x; 1.3263x over previous
import functools

import jax
import jax.numpy as jnp
from jax import lax
from jax.experimental import pallas as pl
from jax.experimental.pallas import tpu as pltpu
from jax.experimental.pallas import tpu_sc as plsc

_CHUNK = 128
_NBUF = 4
_L = 16


def _lookup(idx2, table, v, d):
    n_rows, chunk = idx2.shape
    info = plsc.get_sparse_core_info()
    nw = info.num_cores * info.num_subcores
    n_chunks = n_rows // nw
    b_per_w = n_chunks * chunk
    nbuf = min(_NBUF, n_chunks)
    n_col = d // _L
    n_grp = chunk // _L

    mesh = plsc.VectorSubcoreMesh(core_axis_name="c", subcore_axis_name="s")

    @functools.partial(
        pl.kernel,
        mesh=mesh,
        compiler_params=pltpu.CompilerParams(needs_layout_passes=False),
        out_type=jax.ShapeDtypeStruct((n_rows * chunk, d), jnp.float32),
        scratch_types=(
            [pltpu.VMEM((v * d,), jnp.float32)]
            + [pltpu.VMEM((n_chunks, chunk), jnp.int32)]
            + [pltpu.VMEM((nbuf * chunk, d), jnp.float32)]
            + [pltpu.SemaphoreType.DMA((nbuf,))]
        ),
    )
    def k(table_hbm, idx_hbm, out_hbm, table_v, idx_v, buf, sems):
        wid = lax.axis_index("s") * info.num_cores + lax.axis_index("c")
        pltpu.sync_copy(table_hbm, table_v)
        pltpu.sync_copy(idx_hbm.at[pl.ds(wid * n_chunks, n_chunks)], idx_v)

        cols = [jnp.arange(_L, dtype=jnp.int32) + _L * j for j in range(n_col)]
        lane = [jnp.full((_L,), r, jnp.int32) for r in range(_L)]
        out_base = wid * b_per_w

        def chunk_body(c, _):
            b = lax.rem(c, nbuf)
            slot = b * chunk

            @pl.when(c >= nbuf)
            def _wait_slot():
                pltpu.make_async_copy(
                    buf.at[pl.ds(slot, chunk)],
                    out_hbm.at[pl.ds(out_base, chunk)],
                    sems.at[b],
                ).wait()

            def group_body(g):
                idxvec = idx_v[c, pl.ds(g * _L, _L)]
                row0 = slot + g * _L
                for r in range(_L):
                    iv = idxvec.at[lane[r]].get(mode="promise_in_bounds")
                    ivd = iv * d
                    for j in range(n_col):
                        vals = plsc.load_gather(table_v, [ivd + cols[j]])
                        buf[row0 + r, pl.ds(j * _L, _L)] = vals

            plsc.parallel_loop(0, n_grp, unroll=4)(group_body)
            pltpu.async_copy(
                buf.at[pl.ds(slot, chunk)],
                out_hbm.at[pl.ds(out_base + c * chunk, chunk)],
                sems.at[b],
            )
            return 0

        lax.fori_loop(0, n_chunks, chunk_body, 0)
        for b in range(nbuf):
            pltpu.make_async_copy(
                buf.at[pl.ds(b * chunk, chunk)],
                out_hbm.at[pl.ds(out_base, chunk)],
                sems.at[b],
            ).wait()

    return k(table, idx2)


def kernel(input, table):
    v, d = table.shape
    idx = input.reshape(-1).astype(jnp.int32)
    idx2 = idx.reshape(-1, _CHUNK)
    out = _lookup(idx2, table.astype(jnp.float32).reshape(-1), v, d)
    return out.reshape(input.shape + (d,))
